# baseline (speedup 1.0000x reference)
import functools

import jax
import jax.numpy as jnp
from jax.experimental import pallas as pl
from jax.experimental.pallas import tpu as pltpu

OBS = 64
HID = 64
ACT = 4
LANES = 128
PACK = LANES // ACT
MAX_GRID = 64
MAX_ROWS_PER_STEP = 4096


def _lut_kernel(s_ref, w1b_ref, w2pt_ref, b2bc_ref, o_ref):
    h_t = jax.nn.sigmoid(w1b_ref[...])
    tab_t = (
        jnp.dot(w2pt_ref[...], h_t, preferred_element_type=jnp.float32)
        + b2bc_ref[...]
    )

    s_blk = s_ref[...]
    for j in range(ACT):
        xj = jnp.broadcast_to(tab_t[j:j + 1, :], s_blk.shape)
        o_ref[j::ACT, :] = jnp.take_along_axis(xj, s_blk, axis=1)


@functools.partial(jax.jit, static_argnames=("interpret",))
def _forward(states_i32, w1_t, b1_row, w2_t_pad, b2_row_pad, interpret=False):
    B = states_i32.shape[0]

    w1b = w1_t.T + b1_row.reshape(HID, 1)
    w1b = jnp.pad(w1b, ((0, 0), (0, LANES - OBS)))
    w2pt = w2_t_pad.T
    b2bc = jnp.broadcast_to(b2_row_pad.reshape(LANES, 1), (LANES, LANES))

    n_srows = pl.cdiv(B, LANES)
    grid = MAX_GRID
    sr = pl.cdiv(n_srows, grid)
    if sr > MAX_ROWS_PER_STEP // ACT:
        sr = MAX_ROWS_PER_STEP // ACT
        grid = pl.cdiv(n_srows, sr)
    n_pad = grid * sr * LANES - B
    states_p = jnp.pad(states_i32, (0, n_pad)) if n_pad else states_i32
    s2d = states_p.reshape(grid * sr, LANES)
    pr = sr * ACT

    out_p = pl.pallas_call(
        _lut_kernel,
        out_shape=jax.ShapeDtypeStruct((grid * pr, LANES), jnp.float32),
        grid=(grid,),
        in_specs=[
            pl.BlockSpec((sr, LANES), lambda i: (i, 0)),
            pl.BlockSpec((HID, LANES), lambda i: (0, 0)),
            pl.BlockSpec((LANES, HID), lambda i: (0, 0)),
            pl.BlockSpec((LANES, LANES), lambda i: (0, 0)),
        ],
        out_specs=pl.BlockSpec((pr, LANES), lambda i: (i, 0)),
        compiler_params=pltpu.CompilerParams(
            dimension_semantics=("parallel",)),
        interpret=interpret,
    )(s2d, w1b, w2pt, b2bc)

    n_chunks = out_p.shape[0] // ACT
    out = out_p.reshape(n_chunks, ACT, LANES).transpose(0, 2, 1).reshape(-1, ACT)
    return out[:B] if n_pad else out


def kernel(states_i32, w1_t, b1_row, w2_t_pad, b2_row_pad):
    return _forward(states_i32, w1_t, b1_row, w2_t_pad, b2_row_pad)


# speedup vs baseline: 1.3284x; 1.3284x over previous
import functools

import jax
import jax.numpy as jnp
from jax.experimental import pallas as pl
from jax.experimental.pallas import tpu as pltpu

OBS = 64
HID = 64
ACT = 4
LANES = 128
PACK = LANES // ACT
MAX_GRID = 16
MAX_ROWS_PER_STEP = 8192


def _lut_kernel(s_ref, w1b_ref, w2pt_ref, b2bc_ref, o_ref):
    h_t = jax.nn.sigmoid(w1b_ref[...])
    tab_t = (
        jnp.dot(w2pt_ref[...], h_t, preferred_element_type=jnp.float32)
        + b2bc_ref[...]
    )

    s_blk = s_ref[...]
    for j in range(ACT):
        xj = jnp.broadcast_to(tab_t[j:j + 1, :], s_blk.shape)
        o_ref[j::ACT, :] = jnp.take_along_axis(xj, s_blk, axis=1)


@functools.partial(jax.jit, static_argnames=("interpret",))
def _forward(states_i32, w1_t, b1_row, w2_t_pad, b2_row_pad, interpret=False):
    B = states_i32.shape[0]

    w1b = w1_t.T + b1_row.reshape(HID, 1)
    w1b = jnp.pad(w1b, ((0, 0), (0, LANES - OBS)))
    w2pt = w2_t_pad.T
    b2bc = jnp.broadcast_to(b2_row_pad.reshape(LANES, 1), (LANES, LANES))

    n_srows = pl.cdiv(B, LANES)
    grid = MAX_GRID
    sr = pl.cdiv(n_srows, grid)
    if sr > MAX_ROWS_PER_STEP // ACT:
        sr = MAX_ROWS_PER_STEP // ACT
        grid = pl.cdiv(n_srows, sr)
    n_pad = grid * sr * LANES - B
    states_p = jnp.pad(states_i32, (0, n_pad)) if n_pad else states_i32
    s2d = states_p.reshape(grid * sr, LANES)
    pr = sr * ACT

    out_p = pl.pallas_call(
        _lut_kernel,
        out_shape=jax.ShapeDtypeStruct((grid * pr, LANES), jnp.float32),
        grid=(grid,),
        in_specs=[
            pl.BlockSpec((sr, LANES), lambda i: (i, 0)),
            pl.BlockSpec((HID, LANES), lambda i: (0, 0)),
            pl.BlockSpec((LANES, HID), lambda i: (0, 0)),
            pl.BlockSpec((LANES, LANES), lambda i: (0, 0)),
        ],
        out_specs=pl.BlockSpec((pr, LANES), lambda i: (i, 0)),
        compiler_params=pltpu.CompilerParams(
            dimension_semantics=("parallel",)),
        interpret=interpret,
    )(s2d, w1b, w2pt, b2bc)

    n_chunks = out_p.shape[0] // ACT
    out = out_p.reshape(n_chunks, ACT, LANES).transpose(0, 2, 1).reshape(-1, ACT)
    return out[:B] if n_pad else out


def kernel(states_i32, w1_t, b1_row, w2_t_pad, b2_row_pad):
    return _forward(states_i32, w1_t, b1_row, w2_t_pad, b2_row_pad)


# speedup vs baseline: 1.3290x; 1.0005x over previous
import functools

import jax
import jax.numpy as jnp
from jax.experimental import pallas as pl
from jax.experimental.pallas import tpu as pltpu

OBS = 64
HID = 64
ACT = 4
LANES = 128
PACK = LANES // ACT
MAX_GRID = 8
MAX_ROWS_PER_STEP = 16384


def _lut_kernel(s_ref, w1b_ref, w2pt_ref, b2bc_ref, o_ref):
    h_t = jax.nn.sigmoid(w1b_ref[...])
    tab_t = (
        jnp.dot(w2pt_ref[...], h_t, preferred_element_type=jnp.float32)
        + b2bc_ref[...]
    )

    s_blk = s_ref[...]
    for j in range(ACT):
        xj = jnp.broadcast_to(tab_t[j:j + 1, :], s_blk.shape)
        o_ref[j::ACT, :] = jnp.take_along_axis(xj, s_blk, axis=1)


@functools.partial(jax.jit, static_argnames=("interpret",))
def _forward(states_i32, w1_t, b1_row, w2_t_pad, b2_row_pad, interpret=False):
    B = states_i32.shape[0]

    w1b = w1_t.T + b1_row.reshape(HID, 1)
    w1b = jnp.pad(w1b, ((0, 0), (0, LANES - OBS)))
    w2pt = w2_t_pad.T
    b2bc = jnp.broadcast_to(b2_row_pad.reshape(LANES, 1), (LANES, LANES))

    n_srows = pl.cdiv(B, LANES)
    grid = MAX_GRID
    sr = pl.cdiv(n_srows, grid)
    if sr > MAX_ROWS_PER_STEP // ACT:
        sr = MAX_ROWS_PER_STEP // ACT
        grid = pl.cdiv(n_srows, sr)
    n_pad = grid * sr * LANES - B
    states_p = jnp.pad(states_i32, (0, n_pad)) if n_pad else states_i32
    s2d = states_p.reshape(grid * sr, LANES)
    pr = sr * ACT

    out_p = pl.pallas_call(
        _lut_kernel,
        out_shape=jax.ShapeDtypeStruct((grid * pr, LANES), jnp.float32),
        grid=(grid,),
        in_specs=[
            pl.BlockSpec((sr, LANES), lambda i: (i, 0)),
            pl.BlockSpec((HID, LANES), lambda i: (0, 0)),
            pl.BlockSpec((LANES, HID), lambda i: (0, 0)),
            pl.BlockSpec((LANES, LANES), lambda i: (0, 0)),
        ],
        out_specs=pl.BlockSpec((pr, LANES), lambda i: (i, 0)),
        compiler_params=pltpu.CompilerParams(
            dimension_semantics=("parallel",)),
        interpret=interpret,
    )(s2d, w1b, w2pt, b2bc)

    n_chunks = out_p.shape[0] // ACT
    out = out_p.reshape(n_chunks, ACT, LANES).transpose(0, 2, 1).reshape(-1, ACT)
    return out[:B] if n_pad else out


def kernel(states_i32, w1_t, b1_row, w2_t_pad, b2_row_pad):
    return _forward(states_i32, w1_t, b1_row, w2_t_pad, b2_row_pad)


# speedup vs baseline: 1.3312x; 1.0017x over previous
import functools

import jax
import jax.numpy as jnp
from jax.experimental import pallas as pl
from jax.experimental.pallas import tpu as pltpu

OBS = 64
HID = 64
ACT = 4
LANES = 128
PACK = LANES // ACT
MAX_GRID = 8
MAX_ROWS_PER_STEP = 16384


def _lut_kernel(s_ref, w1b_ref, w2pt_ref, b2bc_ref, o_ref):
    h_t = jax.nn.sigmoid(w1b_ref[...])
    tab_t = (
        jnp.dot(w2pt_ref[...], h_t, preferred_element_type=jnp.float32)
        + b2bc_ref[...]
    )

    s_blk = s_ref[...]
    vals = []
    for j in range(ACT):
        xj = jnp.broadcast_to(tab_t[j:j + 1, :], s_blk.shape)
        vals.append(jnp.take_along_axis(xj, s_blk, axis=1))
    for j in range(ACT):
        o_ref[j::ACT, :] = vals[j]


@functools.partial(jax.jit, static_argnames=("interpret",))
def _forward(states_i32, w1_t, b1_row, w2_t_pad, b2_row_pad, interpret=False):
    B = states_i32.shape[0]

    w1b = w1_t.T + b1_row.reshape(HID, 1)
    w1b = jnp.pad(w1b, ((0, 0), (0, LANES - OBS)))
    w2pt = w2_t_pad.T
    b2bc = jnp.broadcast_to(b2_row_pad.reshape(LANES, 1), (LANES, LANES))

    n_srows = pl.cdiv(B, LANES)
    grid = MAX_GRID
    sr = pl.cdiv(n_srows, grid)
    if sr > MAX_ROWS_PER_STEP // ACT:
        sr = MAX_ROWS_PER_STEP // ACT
        grid = pl.cdiv(n_srows, sr)
    n_pad = grid * sr * LANES - B
    states_p = jnp.pad(states_i32, (0, n_pad)) if n_pad else states_i32
    s2d = states_p.reshape(grid * sr, LANES)
    pr = sr * ACT

    out_p = pl.pallas_call(
        _lut_kernel,
        out_shape=jax.ShapeDtypeStruct((grid * pr, LANES), jnp.float32),
        grid=(grid,),
        in_specs=[
            pl.BlockSpec((sr, LANES), lambda i: (i, 0)),
            pl.BlockSpec((HID, LANES), lambda i: (0, 0)),
            pl.BlockSpec((LANES, HID), lambda i: (0, 0)),
            pl.BlockSpec((LANES, LANES), lambda i: (0, 0)),
        ],
        out_specs=pl.BlockSpec((pr, LANES), lambda i: (i, 0)),
        compiler_params=pltpu.CompilerParams(
            dimension_semantics=("parallel",)),
        interpret=interpret,
    )(s2d, w1b, w2pt, b2bc)

    n_chunks = out_p.shape[0] // ACT
    out = out_p.reshape(n_chunks, ACT, LANES).transpose(0, 2, 1).reshape(-1, ACT)
    return out[:B] if n_pad else out


def kernel(states_i32, w1_t, b1_row, w2_t_pad, b2_row_pad):
    return _forward(states_i32, w1_t, b1_row, w2_t_pad, b2_row_pad)
